# speedup vs baseline: 18.2558x; 18.2558x over previous
import functools

import jax
import jax.numpy as jnp
from jax.experimental import pallas as pl


def _mm_body(x_ref, w_ref, o_ref):
    o_ref[...] = jnp.dot(x_ref[...], w_ref[...],
                         preferred_element_type=jnp.float32)


def _matmul(x, w, bm):
    m, k = x.shape
    n = w.shape[1]
    return pl.pallas_call(
        _mm_body,
        grid=(m // bm,),
        in_specs=[
            pl.BlockSpec((bm, k), lambda i: (i, 0)),
            pl.BlockSpec((k, n), lambda i: (0, 0)),
        ],
        out_specs=pl.BlockSpec((bm, n), lambda i: (i, 0)),
        out_shape=jax.ShapeDtypeStruct((m, n), jnp.float32),
    )(x, w)


def _edge_body(att_ref, gs_ref, gd_ref, ea_ref, o_ref, *, hw):
    gs = gs_ref[...]
    z = gs + gd_ref[...] + ea_ref[...]
    m = jnp.where(z > 0, z, 0.2 * z) * att_ref[...]
    be = gs.shape[0]
    exs = []
    for h in range(4):
        lg = jnp.sum(m[:, h * hw:(h + 1) * hw], axis=1, keepdims=True)
        ex = jnp.exp(lg)
        o_ref[:, h * hw:(h + 1) * hw] = ex * gs[:, h * hw:(h + 1) * hw]
        exs.append(ex)
    lane = jax.lax.broadcasted_iota(jnp.int32, (be, 16), 1)
    exrow = exs[0] * (lane == 0)
    for h in range(1, 4):
        exrow = exrow + exs[h] * (lane == h)
    o_ref[:, 4 * hw:4 * hw + 16] = exrow


def _edge_stage(gs, gd, ea, att, be):
    e, w = gs.shape
    hw = w // 4
    att_flat = att.reshape(1, w)
    return pl.pallas_call(
        functools.partial(_edge_body, hw=hw),
        grid=(e // be,),
        in_specs=[
            pl.BlockSpec((1, w), lambda i: (0, 0)),
            pl.BlockSpec((be, w), lambda i: (i, 0)),
            pl.BlockSpec((be, w), lambda i: (i, 0)),
            pl.BlockSpec((be, w), lambda i: (i, 0)),
        ],
        out_specs=pl.BlockSpec((be, w + 16), lambda i: (i, 0)),
        out_shape=jax.ShapeDtypeStruct((e, w + 16), jnp.float32),
    )(att_flat, gs, gd, ea)


def _norm_body(acc_ref, b_ref, o_ref, *, hw):
    a = acc_ref[...]
    for h in range(4):
        den = a[:, 4 * hw + h:4 * hw + h + 1] + 1e-16
        o_ref[:, h * hw:(h + 1) * hw] = (
            a[:, h * hw:(h + 1) * hw] / den + b_ref[0, h * hw:(h + 1) * hw])


def _normalize(acc, bias, n, bm):
    w = bias.shape[0]
    hw = w // 4
    return pl.pallas_call(
        functools.partial(_norm_body, hw=hw),
        grid=(n // bm,),
        in_specs=[
            pl.BlockSpec((bm, w + 16), lambda i: (i, 0)),
            pl.BlockSpec((1, w), lambda i: (0, 0)),
        ],
        out_specs=pl.BlockSpec((bm, w), lambda i: (i, 0)),
        out_shape=jax.ShapeDtypeStruct((n, w), jnp.float32),
    )(acc[:n], bias.reshape(1, w))


def _segment_rows(wall, dst, n):
    return jax.ops.segment_sum(wall, dst, num_segments=n)


def _gat_layer(x_src_tab, x_dst_tab, src, dst, ea, att, bias, n):
    gs = x_src_tab[src]
    gd = x_dst_tab[dst]
    wall = _edge_stage(gs, gd, ea, att, 512)
    acc = _segment_rows(wall, dst, n)
    return _normalize(acc, bias, n, 512)


def kernel(x_node, x_trace, x_log, node_adj, edge_adj, edge_efea,
           Wl1, Wr1, We1, att1, b1, Wl2, Wr2, We2, att2, b2):
    b, w, npg, d = x_node.shape
    epg = x_trace.shape[2]
    nt = b * w * npg
    et = b * w * epg
    d2 = 2 * d

    node = jnp.concatenate([x_node, x_log], axis=-1).reshape(nt, d2)
    trace = x_trace.reshape(et, d)

    xl1 = _matmul(node, Wl1, 512)
    xr1 = _matmul(node, Wr1, 512)
    ea1 = _matmul(trace, We1, 1024)

    node_out = _gat_layer(xl1, xr1, node_adj[0], node_adj[1], ea1,
                          att1, b1, nt)

    xl2 = _matmul(trace, Wl2, 1024)
    xr2 = _matmul(trace, Wr2, 1024)
    p2 = _matmul(node_out, We2, 512)
    eap = p2[edge_efea]

    trace_out = _gat_layer(xl2, xr2, edge_adj[0], edge_adj[1], eap,
                           att2, b2, et)

    xn = node_out[:, :d].reshape(b, w, npg, d)
    xt = trace_out.reshape(b, w, epg, d)
    xl = node_out[:, d:].reshape(b, w, npg, d)
    return (xn, xt, xl)


# speedup vs baseline: 25.4291x; 1.3929x over previous
import functools

import jax
import jax.numpy as jnp
from jax import lax
from jax.experimental import pallas as pl
from jax.experimental.pallas import tpu as pltpu
from jax.experimental.pallas import tpu_sc as plsc

_NC = 2
_NS = 16
_NW = _NC * _NS


def _sc_gather(table, idx, block=128):
    v, d = table.shape
    b = idx.shape[0]
    bw = b // _NW
    mesh = plsc.VectorSubcoreMesh(core_axis_name="c", subcore_axis_name="s")

    @functools.partial(
        pl.kernel, mesh=mesh,
        out_type=jax.ShapeDtypeStruct((b, d), jnp.float32),
        scratch_types=[
            pltpu.VMEM((block,), jnp.int32),
            pltpu.VMEM((block, d), jnp.float32),
            pltpu.SemaphoreType.DMA,
        ],
    )
    def k(table_hbm, idx_hbm, out_hbm, idx_v, rows_v, sem):
        wid = lax.axis_index("s") * _NC + lax.axis_index("c")
        base = wid * bw

        @pl.loop(0, bw, step=block)
        def _(off):
            pltpu.sync_copy(idx_hbm.at[pl.ds(base + off, block)], idx_v)
            pltpu.async_copy(table_hbm.at[idx_v], rows_v, sem).wait()
            pltpu.sync_copy(rows_v, out_hbm.at[pl.ds(base + off, block)])

    return k(table, idx)


def _mm_body(x_ref, w_ref, o_ref):
    o_ref[...] = jnp.dot(x_ref[...], w_ref[...],
                         preferred_element_type=jnp.float32)


def _matmul(x, w, bm):
    m, k = x.shape
    n = w.shape[1]
    return pl.pallas_call(
        _mm_body,
        grid=(m // bm,),
        in_specs=[
            pl.BlockSpec((bm, k), lambda i: (i, 0)),
            pl.BlockSpec((k, n), lambda i: (0, 0)),
        ],
        out_specs=pl.BlockSpec((bm, n), lambda i: (i, 0)),
        out_shape=jax.ShapeDtypeStruct((m, n), jnp.float32),
    )(x, w)


def _edge_body(att_ref, gs_ref, gd_ref, ea_ref, o_ref, *, hw):
    gs = gs_ref[...]
    z = gs + gd_ref[...] + ea_ref[...]
    m = jnp.where(z > 0, z, 0.2 * z) * att_ref[...]
    be = gs.shape[0]
    exs = []
    for h in range(4):
        lg = jnp.sum(m[:, h * hw:(h + 1) * hw], axis=1, keepdims=True)
        ex = jnp.exp(lg)
        o_ref[:, h * hw:(h + 1) * hw] = ex * gs[:, h * hw:(h + 1) * hw]
        exs.append(ex)
    lane = jax.lax.broadcasted_iota(jnp.int32, (be, 16), 1)
    exrow = exs[0] * (lane == 0)
    for h in range(1, 4):
        exrow = exrow + exs[h] * (lane == h)
    o_ref[:, 4 * hw:4 * hw + 16] = exrow


def _edge_stage(gs, gd, ea, att, be):
    e, w = gs.shape
    hw = w // 4
    att_flat = att.reshape(1, w)
    return pl.pallas_call(
        functools.partial(_edge_body, hw=hw),
        grid=(e // be,),
        in_specs=[
            pl.BlockSpec((1, w), lambda i: (0, 0)),
            pl.BlockSpec((be, w), lambda i: (i, 0)),
            pl.BlockSpec((be, w), lambda i: (i, 0)),
            pl.BlockSpec((be, w), lambda i: (i, 0)),
        ],
        out_specs=pl.BlockSpec((be, w + 16), lambda i: (i, 0)),
        out_shape=jax.ShapeDtypeStruct((e, w + 16), jnp.float32),
    )(att_flat, gs, gd, ea)


def _norm_body(acc_ref, b_ref, o_ref, *, hw):
    a = acc_ref[...]
    for h in range(4):
        den = a[:, 4 * hw + h:4 * hw + h + 1] + 1e-16
        o_ref[:, h * hw:(h + 1) * hw] = (
            a[:, h * hw:(h + 1) * hw] / den + b_ref[0, h * hw:(h + 1) * hw])


def _normalize(acc, bias, n, bm):
    w = bias.shape[0]
    hw = w // 4
    return pl.pallas_call(
        functools.partial(_norm_body, hw=hw),
        grid=(n // bm,),
        in_specs=[
            pl.BlockSpec((bm, w + 16), lambda i: (i, 0)),
            pl.BlockSpec((1, w), lambda i: (0, 0)),
        ],
        out_specs=pl.BlockSpec((bm, w), lambda i: (i, 0)),
        out_shape=jax.ShapeDtypeStruct((n, w), jnp.float32),
    )(acc[:n], bias.reshape(1, w))


def _segment_rows(wall, dst, n):
    return jax.ops.segment_sum(wall, dst, num_segments=n)


def _gat_layer(x_src_tab, x_dst_tab, src, dst, ea, att, bias, n):
    gs = _sc_gather(x_src_tab, src)
    gd = _sc_gather(x_dst_tab, dst)
    wall = _edge_stage(gs, gd, ea, att, 512)
    acc = _segment_rows(wall, dst, n)
    return _normalize(acc, bias, n, 512)


def kernel(x_node, x_trace, x_log, node_adj, edge_adj, edge_efea,
           Wl1, Wr1, We1, att1, b1, Wl2, Wr2, We2, att2, b2):
    b, w, npg, d = x_node.shape
    epg = x_trace.shape[2]
    nt = b * w * npg
    et = b * w * epg
    d2 = 2 * d

    node = jnp.concatenate([x_node, x_log], axis=-1).reshape(nt, d2)
    trace = x_trace.reshape(et, d)

    xl1 = _matmul(node, Wl1, 512)
    xr1 = _matmul(node, Wr1, 512)
    ea1 = _matmul(trace, We1, 1024)

    node_out = _gat_layer(xl1, xr1, node_adj[0], node_adj[1], ea1,
                          att1, b1, nt)

    xl2 = _matmul(trace, Wl2, 1024)
    xr2 = _matmul(trace, Wr2, 1024)
    p2 = _matmul(node_out, We2, 512)
    eap = _sc_gather(p2, edge_efea)

    trace_out = _gat_layer(xl2, xr2, edge_adj[0], edge_adj[1], eap,
                           att2, b2, et)

    xn = node_out[:, :d].reshape(b, w, npg, d)
    xt = trace_out.reshape(b, w, epg, d)
    xl = node_out[:, d:].reshape(b, w, npg, d)
    return (xn, xt, xl)


# speedup vs baseline: 28.2831x; 1.1122x over previous
import dataclasses
import functools

import jax
import jax.numpy as jnp
from jax import lax
from jax.experimental import pallas as pl
from jax.experimental.pallas import tpu as pltpu
from jax.experimental.pallas import tpu_sc as plsc

_NC = 2
_NS = 16
_NW = _NC * _NS


def _sc_compiler_params():
    cp = pltpu.CompilerParams()
    if "needs_layout_passes" in pltpu.CompilerParams.__dataclass_fields__:
        cp = dataclasses.replace(cp, needs_layout_passes=False)
    return cp


def _sc_gather(table, idx, block=128):
    v, d = table.shape
    b = idx.shape[0]
    bw = b // _NW
    mesh = plsc.VectorSubcoreMesh(core_axis_name="c", subcore_axis_name="s")

    @functools.partial(
        pl.kernel, mesh=mesh,
        out_type=jax.ShapeDtypeStruct((b, d), jnp.float32),
        scratch_types=[
            pltpu.VMEM((block,), jnp.int32),
            pltpu.VMEM((block, d), jnp.float32),
            pltpu.SemaphoreType.DMA,
        ],
    )
    def k(table_hbm, idx_hbm, out_hbm, idx_v, rows_v, sem):
        wid = lax.axis_index("s") * _NC + lax.axis_index("c")
        base = wid * bw

        @pl.loop(0, bw, step=block)
        def _(off):
            pltpu.sync_copy(idx_hbm.at[pl.ds(base + off, block)], idx_v)
            pltpu.async_copy(table_hbm.at[idx_v], rows_v, sem).wait()
            pltpu.sync_copy(rows_v, out_hbm.at[pl.ds(base + off, block)])

    return k(table, idx)


def _mm_body(x_ref, w_ref, o_ref):
    o_ref[...] = jnp.dot(x_ref[...], w_ref[...],
                         preferred_element_type=jnp.float32)


def _matmul(x, w, bm):
    m, k = x.shape
    n = w.shape[1]
    return pl.pallas_call(
        _mm_body,
        grid=(m // bm,),
        in_specs=[
            pl.BlockSpec((bm, k), lambda i: (i, 0)),
            pl.BlockSpec((k, n), lambda i: (0, 0)),
        ],
        out_specs=pl.BlockSpec((bm, n), lambda i: (i, 0)),
        out_shape=jax.ShapeDtypeStruct((m, n), jnp.float32),
    )(x, w)


def _edge_body(att_ref, gs_ref, gd_ref, ea_ref, o_ref, *, hw):
    gs = gs_ref[...]
    z = gs + gd_ref[...] + ea_ref[...]
    m = jnp.where(z > 0, z, 0.2 * z) * att_ref[...]
    be = gs.shape[0]
    exs = []
    for h in range(4):
        lg = jnp.sum(m[:, h * hw:(h + 1) * hw], axis=1, keepdims=True)
        ex = jnp.exp(lg)
        p, off = (h * hw) // 128, (h * hw) % 128
        o_ref[p, :, off:off + hw] = ex * gs[:, h * hw:(h + 1) * hw]
        exs.append(ex)
    lane = jax.lax.broadcasted_iota(jnp.int32, (be, 128), 1)
    exrow = exs[0] * (lane == 0)
    for h in range(1, 4):
        exrow = exrow + exs[h] * (lane == h)
    o_ref[o_ref.shape[0] - 1, :, :] = exrow


def _edge_stage(gs, gd, ea, att, be):
    e, w = gs.shape
    hw = w // 4
    p = w // 128 + 1
    att_flat = att.reshape(1, w)
    return pl.pallas_call(
        functools.partial(_edge_body, hw=hw),
        grid=(e // be,),
        in_specs=[
            pl.BlockSpec((1, w), lambda i: (0, 0)),
            pl.BlockSpec((be, w), lambda i: (i, 0)),
            pl.BlockSpec((be, w), lambda i: (i, 0)),
            pl.BlockSpec((be, w), lambda i: (i, 0)),
        ],
        out_specs=pl.BlockSpec((p, be, 128), lambda i: (0, i, 0)),
        out_shape=jax.ShapeDtypeStruct((p, e, 128), jnp.float32),
    )(att_flat, gs, gd, ea)


def _norm_body(*refs, hw):
    *num_refs, den_ref, b_ref, o_ref = refs
    den = den_ref[...]
    for h in range(4):
        dh = den[:, h:h + 1] + 1e-16
        p, off = (h * hw) // 128, (h * hw) % 128
        o_ref[:, h * hw:(h + 1) * hw] = (
            num_refs[p][:, off:off + hw] / dh + b_ref[0, h * hw:(h + 1) * hw])


def _normalize(nums, den, bias, n, bm):
    w = bias.shape[0]
    hw = w // 4
    specs = [pl.BlockSpec((bm, 128), lambda i: (i, 0)) for _ in nums]
    specs.append(pl.BlockSpec((bm, 128), lambda i: (i, 0)))
    specs.append(pl.BlockSpec((1, w), lambda i: (0, 0)))
    return pl.pallas_call(
        functools.partial(_norm_body, hw=hw),
        grid=(n // bm,),
        in_specs=specs,
        out_specs=pl.BlockSpec((bm, w), lambda i: (i, 0)),
        out_shape=jax.ShapeDtypeStruct((n, w), jnp.float32),
    )(*[x[:n] for x in nums], den[:n], bias.reshape(1, w))


def _segment_rows(tbl, parts, dst, n, chunk, block=64):
    e = dst.shape[0]
    nchunk = -(-n // chunk)
    if nchunk % _NC:
        nchunk += 1
    cpc = nchunk // _NC
    npad = nchunk * chunk
    eps = e // _NS
    dp = 2048
    rpw = chunk // _NS
    trash = chunk
    mesh = plsc.VectorSubcoreMesh(core_axis_name="c", subcore_axis_name="s")
    zeros = jnp.zeros((chunk, 128), jnp.float32)

    @functools.partial(
        pl.kernel, mesh=mesh,
        out_type=jax.ShapeDtypeStruct((parts, npad, 128), jnp.float32),
        scratch_types=[
            pltpu.VMEM((dp,), jnp.int32),
            pltpu.VMEM((eps,), jnp.int32),
            pltpu.VMEM((block,), jnp.int32),
            pltpu.VMEM((block,), jnp.int32),
            pltpu.VMEM((block, 128), jnp.float32),
            pltpu.VMEM_SHARED((chunk + 8, 128), jnp.float32),
            pltpu.SemaphoreType.DMA,
        ],
        compiler_params=_sc_compiler_params(),
    )
    def k(w_hbm, dst_hbm, z_hbm, out_hbm, dst_v, list_v, eid_v, ld_v,
          rows_t, acc, sem):
        cid = lax.axis_index("c")
        sid = lax.axis_index("s")
        ebase = sid * eps
        iota = lax.iota(jnp.int32, 16)
        pltpu.sync_copy(z_hbm.at[pl.ds(sid * rpw, rpw)],
                        acc.at[pl.ds(sid * rpw, rpw)])

        @pl.loop(0, cpc)
        def _chunk(cc):
            lo = (cid * cpc + cc) * chunk

            def piece(pp, cnt0):
                pltpu.sync_copy(dst_hbm.at[pl.ds(ebase + pp * dp, dp)], dst_v)

                def compact(j, cnt):
                    dstv = dst_v[pl.ds(j * 16, 16)]
                    inb = (dstv >= lo) & (dstv < lo + chunk)
                    inbi = jnp.where(inb, 1, 0)
                    pos = jnp.maximum(cnt + plsc.cumsum(inbi) - 1, 0)
                    packed = ((dstv - lo) << 15) | (pp * dp + j * 16 + iota)
                    plsc.store_scatter(list_v, [pos], packed, mask=inb)
                    return cnt + jnp.sum(inbi)

                return lax.fori_loop(0, dp // 16, compact, cnt0)

            cnt = lax.fori_loop(0, eps // dp, piece, jnp.int32(0))
            nblk = (cnt + block - 1) // block

            def pad(j, _):
                pp = (cnt // 16) * 16 + j * 16 + iota
                m = (pp >= cnt) & (pp < nblk * block)
                plsc.store_scatter(list_v, [jnp.maximum(pp, 0)],
                                   jnp.full((16,), trash << 15, jnp.int32),
                                   mask=m)
                return 0

            lax.fori_loop(0, block // 16, pad, 0)

            for part in range(parts):
                peb = part * e + ebase
                plsc.subcore_barrier()

                def blk(b, _):
                    for t in range(block // 16):
                        v = list_v[pl.ds(b * block + t * 16, 16)]
                        eid_v[pl.ds(t * 16, 16)] = peb + (v & 0x7FFF)
                        ld_v[pl.ds(t * 16, 16)] = (
                            lax.shift_right_logical(v, 15))
                    pltpu.async_copy(w_hbm.at[eid_v], rows_t, sem).wait()
                    pltpu.async_copy(rows_t, acc.at[ld_v], sem,
                                     add=True).wait()
                    return 0

                lax.fori_loop(0, nblk, blk, 0)
                plsc.subcore_barrier()

                pltpu.sync_copy(acc.at[pl.ds(sid * rpw, rpw)],
                                out_hbm.at[part, pl.ds(lo + sid * rpw, rpw)])
                pltpu.sync_copy(z_hbm.at[pl.ds(sid * rpw, rpw)],
                                acc.at[pl.ds(sid * rpw, rpw)])

    out = k(tbl, dst, zeros)
    return [out[i] for i in range(parts)]


def _gat_layer(x_src_tab, x_dst_tab, src, dst, ea, att, bias, n, chunk):
    gs = _sc_gather(x_src_tab, src)
    gd = _sc_gather(x_dst_tab, dst)
    wall = _edge_stage(gs, gd, ea, att, 512)
    parts = wall.shape[0]
    outs = _segment_rows(wall.reshape(parts * wall.shape[1], 128), parts,
                         dst, n, chunk)
    return _normalize(outs[:-1], outs[-1], bias, n, 512)


def kernel(x_node, x_trace, x_log, node_adj, edge_adj, edge_efea,
           Wl1, Wr1, We1, att1, b1, Wl2, Wr2, We2, att2, b2):
    b, w, npg, d = x_node.shape
    epg = x_trace.shape[2]
    nt = b * w * npg
    et = b * w * epg
    d2 = 2 * d

    node = jnp.concatenate([x_node, x_log], axis=-1).reshape(nt, d2)
    trace = x_trace.reshape(et, d)

    xl1 = _matmul(node, Wl1, 512)
    xr1 = _matmul(node, Wr1, 512)
    ea1 = _matmul(trace, We1, 1024)

    node_out = _gat_layer(xl1, xr1, node_adj[0], node_adj[1], ea1,
                          att1, b1, nt, 5120)

    xl2 = _matmul(trace, Wl2, 1024)
    xr2 = _matmul(trace, Wr2, 1024)
    p2 = _matmul(node_out, We2, 512)
    eap = _sc_gather(p2, edge_efea)

    trace_out = _gat_layer(xl2, xr2, edge_adj[0], edge_adj[1], eap,
                           att2, b2, et, 12288)

    xn = node_out[:, :d].reshape(b, w, npg, d)
    xt = trace_out.reshape(b, w, epg, d)
    xl = node_out[:, d:].reshape(b, w, npg, d)
    return (xn, xt, xl)


# speedup vs baseline: 32.2806x; 1.1413x over previous
import dataclasses
import functools

import jax
import jax.numpy as jnp
from jax import lax
from jax.experimental import pallas as pl
from jax.experimental.pallas import tpu as pltpu
from jax.experimental.pallas import tpu_sc as plsc

_NC = 2
_NS = 16
_NW = _NC * _NS


def _sc_compiler_params():
    cp = pltpu.CompilerParams()
    if "needs_layout_passes" in pltpu.CompilerParams.__dataclass_fields__:
        cp = dataclasses.replace(cp, needs_layout_passes=False)
    return cp


def _sc_gather(table, idx, nbuf=4):
    v, d = table.shape
    b = idx.shape[0]
    bw = b // _NW
    block = 128 if d <= 128 else 64
    nb = bw // block
    assert nb % nbuf == 0
    mesh = plsc.VectorSubcoreMesh(core_axis_name="c", subcore_axis_name="s")

    @functools.partial(
        pl.kernel, mesh=mesh,
        out_type=jax.ShapeDtypeStruct((b, d), jnp.float32),
        scratch_types=[
            pltpu.VMEM((nbuf, block), jnp.int32),
            pltpu.VMEM((nbuf, block, d), jnp.float32),
        ] + [pltpu.SemaphoreType.DMA] * nbuf,
    )
    def k(table_hbm, idx_hbm, out_hbm, idx_v, rows_v, *sems):
        wid = lax.axis_index("s") * _NC + lax.axis_index("c")
        base = wid * bw
        for u in range(nbuf):
            pltpu.sync_copy(idx_hbm.at[pl.ds(base + u * block, block)],
                            idx_v.at[u])
            pltpu.async_copy(table_hbm.at[idx_v.at[u]], rows_v.at[u], sems[u])

        @pl.loop(0, nb, step=nbuf)
        def _(i):
            for u in range(nbuf):
                bb = i + u
                pltpu.make_async_copy(table_hbm.at[idx_v.at[u]],
                                      rows_v.at[u], sems[u]).wait()
                pltpu.sync_copy(rows_v.at[u],
                                out_hbm.at[pl.ds(base + bb * block, block)])
                nxt = bb + nbuf

                @pl.when(nxt < nb)
                def _():
                    pltpu.sync_copy(
                        idx_hbm.at[pl.ds(base + nxt * block, block)],
                        idx_v.at[u])
                    pltpu.async_copy(table_hbm.at[idx_v.at[u]],
                                     rows_v.at[u], sems[u])

    return k(table, idx)


def _mm_body(x_ref, w_ref, o_ref):
    o_ref[...] = jnp.dot(x_ref[...], w_ref[...],
                         preferred_element_type=jnp.float32)


def _matmul(x, w, bm):
    m, k = x.shape
    n = w.shape[1]
    return pl.pallas_call(
        _mm_body,
        grid=(m // bm,),
        in_specs=[
            pl.BlockSpec((bm, k), lambda i: (i, 0)),
            pl.BlockSpec((k, n), lambda i: (0, 0)),
        ],
        out_specs=pl.BlockSpec((bm, n), lambda i: (i, 0)),
        out_shape=jax.ShapeDtypeStruct((m, n), jnp.float32),
    )(x, w)


def _edge_body(att_ref, gs_ref, gd_ref, ea_ref, o_ref, *, hw):
    gs = gs_ref[...]
    z = gs + gd_ref[...] + ea_ref[...]
    m = jnp.where(z > 0, z, 0.2 * z) * att_ref[...]
    be = gs.shape[0]
    exs = []
    for h in range(4):
        lg = jnp.sum(m[:, h * hw:(h + 1) * hw], axis=1, keepdims=True)
        ex = jnp.exp(lg)
        p, off = (h * hw) // 128, (h * hw) % 128
        o_ref[p, :, off:off + hw] = ex * gs[:, h * hw:(h + 1) * hw]
        exs.append(ex)
    lane = jax.lax.broadcasted_iota(jnp.int32, (be, 128), 1)
    exrow = exs[0] * (lane == 0)
    for h in range(1, 4):
        exrow = exrow + exs[h] * (lane == h)
    o_ref[o_ref.shape[0] - 1, :, :] = exrow


def _edge_stage(gs, gd, ea, att, be):
    e, w = gs.shape
    hw = w // 4
    p = w // 128 + 1
    att_flat = att.reshape(1, w)
    return pl.pallas_call(
        functools.partial(_edge_body, hw=hw),
        grid=(e // be,),
        in_specs=[
            pl.BlockSpec((1, w), lambda i: (0, 0)),
            pl.BlockSpec((be, w), lambda i: (i, 0)),
            pl.BlockSpec((be, w), lambda i: (i, 0)),
            pl.BlockSpec((be, w), lambda i: (i, 0)),
        ],
        out_specs=pl.BlockSpec((p, be, 128), lambda i: (0, i, 0)),
        out_shape=jax.ShapeDtypeStruct((p, e, 128), jnp.float32),
    )(att_flat, gs, gd, ea)


def _norm_body(*refs, hw):
    *num_refs, den_ref, b_ref, o_ref = refs
    den = den_ref[...]
    for h in range(4):
        dh = den[:, h:h + 1] + 1e-16
        p, off = (h * hw) // 128, (h * hw) % 128
        o_ref[:, h * hw:(h + 1) * hw] = (
            num_refs[p][:, off:off + hw] / dh + b_ref[0, h * hw:(h + 1) * hw])


def _normalize(nums, den, bias, n, bm):
    w = bias.shape[0]
    hw = w // 4
    specs = [pl.BlockSpec((bm, 128), lambda i: (i, 0)) for _ in nums]
    specs.append(pl.BlockSpec((bm, 128), lambda i: (i, 0)))
    specs.append(pl.BlockSpec((1, w), lambda i: (0, 0)))
    return pl.pallas_call(
        functools.partial(_norm_body, hw=hw),
        grid=(n // bm,),
        in_specs=specs,
        out_specs=pl.BlockSpec((bm, w), lambda i: (i, 0)),
        out_shape=jax.ShapeDtypeStruct((n, w), jnp.float32),
    )(*[x[:n] for x in nums], den[:n], bias.reshape(1, w))


def _segment_rows(tbl, parts, dst, n, chunk, block=64):
    e = dst.shape[0]
    nchunk = -(-n // chunk)
    if nchunk % _NC:
        nchunk += 1
    cpc = nchunk // _NC
    npad = nchunk * chunk
    eps = e // _NS
    dp = 2048
    rpw = chunk // _NS
    trash = chunk
    mesh = plsc.VectorSubcoreMesh(core_axis_name="c", subcore_axis_name="s")
    zeros = jnp.zeros((chunk, 128), jnp.float32)

    @functools.partial(
        pl.kernel, mesh=mesh,
        out_type=jax.ShapeDtypeStruct((parts, npad, 128), jnp.float32),
        scratch_types=[
            pltpu.VMEM((dp,), jnp.int32),
            pltpu.VMEM((eps,), jnp.int32),
            pltpu.VMEM((2, block), jnp.int32),
            pltpu.VMEM((2, block), jnp.int32),
            pltpu.VMEM((2, block, 128), jnp.float32),
            pltpu.VMEM_SHARED((chunk + 8, 128), jnp.float32),
            pltpu.SemaphoreType.DMA,
            pltpu.SemaphoreType.DMA,
            pltpu.SemaphoreType.DMA,
            pltpu.SemaphoreType.DMA,
        ],
        compiler_params=_sc_compiler_params(),
    )
    def k(w_hbm, dst_hbm, z_hbm, out_hbm, dst_v, list_v, eid_v, ld_v,
          rows_t, acc, semg0, semg1, sema0, sema1):
        semg = (semg0, semg1)
        sema = (sema0, sema1)
        cid = lax.axis_index("c")
        sid = lax.axis_index("s")
        ebase = sid * eps
        iota = lax.iota(jnp.int32, 16)
        pltpu.sync_copy(z_hbm.at[pl.ds(sid * rpw, rpw)],
                        acc.at[pl.ds(sid * rpw, rpw)])

        @pl.loop(0, cpc)
        def _chunk(cc):
            lo = (cid * cpc + cc) * chunk

            def piece(pp, cnt0):
                pltpu.sync_copy(dst_hbm.at[pl.ds(ebase + pp * dp, dp)], dst_v)

                def compact(j, cnt):
                    dstv = dst_v[pl.ds(j * 16, 16)]
                    inb = (dstv >= lo) & (dstv < lo + chunk)
                    inbi = jnp.where(inb, 1, 0)
                    pos = jnp.maximum(cnt + plsc.cumsum(inbi) - 1, 0)
                    packed = ((dstv - lo) << 15) | (pp * dp + j * 16 + iota)
                    plsc.store_scatter(list_v, [pos], packed, mask=inb)
                    return cnt + jnp.sum(inbi)

                return lax.fori_loop(0, dp // 16, compact, cnt0)

            cnt = lax.fori_loop(0, eps // dp, piece, jnp.int32(0))
            nblk2 = ((cnt + 2 * block - 1) // (2 * block)) * 2

            def pad(j, _):
                pp = (cnt // 16) * 16 + j * 16 + iota
                m = (pp >= cnt) & (pp < nblk2 * block)
                plsc.store_scatter(list_v, [jnp.maximum(pp, 0)],
                                   jnp.full((16,), trash << 15, jnp.int32),
                                   mask=m)
                return 0

            lax.fori_loop(0, (2 * block) // 16 + 1, pad, 0)

            for part in range(parts):
                peb = part * e + ebase
                plsc.subcore_barrier()

                def pair(j, _):
                    for u in range(2):
                        b = 2 * j + u

                        @pl.when(j > 0)
                        def _():
                            pltpu.make_async_copy(
                                rows_t.at[u], acc.at[ld_v.at[u]],
                                sema[u]).wait()

                        for t in range(block // 16):
                            v = list_v[pl.ds(b * block + t * 16, 16)]
                            eid_v.at[u][pl.ds(t * 16, 16)] = peb + (v & 0x7FFF)
                            ld_v.at[u][pl.ds(t * 16, 16)] = (
                                lax.shift_right_logical(v, 15))
                        pltpu.async_copy(w_hbm.at[eid_v.at[u]],
                                         rows_t.at[u], semg[u])
                    for u in range(2):
                        pltpu.make_async_copy(w_hbm.at[eid_v.at[u]],
                                              rows_t.at[u], semg[u]).wait()
                        pltpu.async_copy(rows_t.at[u], acc.at[ld_v.at[u]],
                                         sema[u], add=True)
                    return 0

                lax.fori_loop(0, nblk2 // 2, pair, 0)

                @pl.when(nblk2 > 0)
                def _():
                    for u in range(2):
                        pltpu.make_async_copy(rows_t.at[u],
                                              acc.at[ld_v.at[u]],
                                              sema[u]).wait()

                plsc.subcore_barrier()

                pltpu.sync_copy(acc.at[pl.ds(sid * rpw, rpw)],
                                out_hbm.at[part, pl.ds(lo + sid * rpw, rpw)])
                pltpu.sync_copy(z_hbm.at[pl.ds(sid * rpw, rpw)],
                                acc.at[pl.ds(sid * rpw, rpw)])

    out = k(tbl, dst, zeros)
    return [out[i] for i in range(parts)]


def _gat_layer(x_src_tab, x_dst_tab, src, dst, ea, att, bias, n, chunk):
    gs = _sc_gather(x_src_tab, src)
    gd = _sc_gather(x_dst_tab, dst)
    wall = _edge_stage(gs, gd, ea, att, 512)
    parts = wall.shape[0]
    outs = _segment_rows(wall.reshape(parts * wall.shape[1], 128), parts,
                         dst, n, chunk)
    return _normalize(outs[:-1], outs[-1], bias, n, 512)


def kernel(x_node, x_trace, x_log, node_adj, edge_adj, edge_efea,
           Wl1, Wr1, We1, att1, b1, Wl2, Wr2, We2, att2, b2):
    b, w, npg, d = x_node.shape
    epg = x_trace.shape[2]
    nt = b * w * npg
    et = b * w * epg
    d2 = 2 * d

    node = jnp.concatenate([x_node, x_log], axis=-1).reshape(nt, d2)
    trace = x_trace.reshape(et, d)

    xl1 = _matmul(node, Wl1, 512)
    xr1 = _matmul(node, Wr1, 512)
    ea1 = _matmul(trace, We1, 1024)

    node_out = _gat_layer(xl1, xr1, node_adj[0], node_adj[1], ea1,
                          att1, b1, nt, 5120)

    xl2 = _matmul(trace, Wl2, 1024)
    xr2 = _matmul(trace, Wr2, 1024)
    p2 = _matmul(node_out, We2, 512)
    eap = _sc_gather(p2, edge_efea)

    trace_out = _gat_layer(xl2, xr2, edge_adj[0], edge_adj[1], eap,
                           att2, b2, et, 10240)

    xn = node_out[:, :d].reshape(b, w, npg, d)
    xt = trace_out.reshape(b, w, epg, d)
    xl = node_out[:, d:].reshape(b, w, npg, d)
    return (xn, xt, xl)
